# initial kernel scaffold (unmeasured)
import jax
import jax.numpy as jnp
from jax import lax
from jax.experimental import pallas as pl
from jax.experimental.pallas import tpu as pltpu

N_DEV = 32


def kernel(Q, K, V):
    b, s, h, d = Q.shape
    bh = b * h
    scale = d ** -0.5

    q = jnp.transpose(Q, (0, 2, 1, 3)).reshape(bh, s, d)
    k = jnp.transpose(K, (0, 2, 1, 3)).reshape(bh, s, d)
    v = jnp.transpose(V, (0, 2, 1, 3)).reshape(bh, s, d)

    def body(q_ref, k_ref, v_ref, out_ref,
             kc, vc, acc, m_ref, l_ref,
             ksend, krecv, vsend, vrecv, credit):
        my_pos = lax.axis_index("i")
        left = lax.rem(my_pos + N_DEV - 1, N_DEV)
        right = lax.rem(my_pos + 1, N_DEV)

        barrier = pltpu.get_barrier_semaphore()
        for nbr in (left, right):
            pl.semaphore_signal(
                barrier, inc=1,
                device_id=(nbr,), device_id_type=pl.DeviceIdType.MESH,
            )
        pl.semaphore_wait(barrier, 2)

        m_ref[...] = jnp.full((bh, s), -1e30, dtype=jnp.float32)
        l_ref[...] = jnp.zeros((bh, s), dtype=jnp.float32)
        acc[...] = jnp.zeros((bh, s, d), dtype=jnp.float32)

        kc[0] = k_ref[...]
        vc[0] = v_ref[...]

        def flash_chunk(slot):
            def bh_body(i, _):
                qi = q_ref[i]
                ki = kc[slot, i]
                vi = vc[slot, i]
                sij = lax.dot_general(
                    qi, ki, (((1,), (1,)), ((), ())),
                    preferred_element_type=jnp.float32,
                ) * scale
                m_old = m_ref[i]
                m_new = jnp.maximum(m_old, jnp.max(sij, axis=1))
                p = jnp.exp(sij - m_new[:, None])
                alpha = jnp.exp(m_old - m_new)
                l_ref[i] = l_ref[i] * alpha + jnp.sum(p, axis=1)
                pv = lax.dot_general(
                    p, vi, (((1,), (0,)), ((), ())),
                    preferred_element_type=jnp.float32,
                )
                acc[i] = acc[i] * alpha[:, None] + pv
                m_ref[i] = m_new
                return 0
            lax.fori_loop(0, bh, bh_body, 0)

        for hop in range(N_DEV - 1):
            cur = hop % 2
            nxt = (hop + 1) % 2
            if hop >= 1:
                pl.semaphore_wait(credit, 1)
            krdma = pltpu.make_async_remote_copy(
                src_ref=kc.at[cur], dst_ref=kc.at[nxt],
                send_sem=ksend.at[cur], recv_sem=krecv.at[nxt],
                device_id=(right,), device_id_type=pl.DeviceIdType.MESH,
            )
            vrdma = pltpu.make_async_remote_copy(
                src_ref=vc.at[cur], dst_ref=vc.at[nxt],
                send_sem=vsend.at[cur], recv_sem=vrecv.at[nxt],
                device_id=(right,), device_id_type=pl.DeviceIdType.MESH,
            )
            krdma.start()
            vrdma.start()
            flash_chunk(cur)
            krdma.wait()
            vrdma.wait()
            if hop < N_DEV - 2:
                pl.semaphore_signal(
                    credit, inc=1,
                    device_id=(left,), device_id_type=pl.DeviceIdType.MESH,
                )
        flash_chunk((N_DEV - 1) % 2)

        def norm_body(i, _):
            out_ref[i] = acc[i] / l_ref[i][:, None]
            return 0
        lax.fori_loop(0, bh, norm_body, 0)

    out = pl.pallas_call(
        body,
        out_shape=jax.ShapeDtypeStruct((bh, s, d), jnp.float32),
        in_specs=[pl.BlockSpec(memory_space=pltpu.VMEM)] * 3,
        out_specs=pl.BlockSpec(memory_space=pltpu.VMEM),
        scratch_shapes=[
            pltpu.VMEM((2, bh, s, d), jnp.float32),
            pltpu.VMEM((2, bh, s, d), jnp.float32),
            pltpu.VMEM((bh, s, d), jnp.float32),
            pltpu.VMEM((bh, s), jnp.float32),
            pltpu.VMEM((bh, s), jnp.float32),
            pltpu.SemaphoreType.DMA((2,)),
            pltpu.SemaphoreType.DMA((2,)),
            pltpu.SemaphoreType.DMA((2,)),
            pltpu.SemaphoreType.DMA((2,)),
            pltpu.SemaphoreType.REGULAR,
        ],
        compiler_params=pltpu.CompilerParams(collective_id=0),
    )(q, k, v)

    return jnp.transpose(out.reshape(b, h, s, d), (0, 2, 1, 3))


# baseline (device time: 2943089 ns/iter reference)
import jax
import jax.numpy as jnp
from jax import lax
from jax.experimental import pallas as pl
from jax.experimental.pallas import tpu as pltpu

N_DEV = 32


def kernel(Q, K, V):
    b, s, h, d = Q.shape
    bh = b * h
    scale = d ** -0.5

    q = jnp.transpose(Q, (0, 2, 1, 3)).reshape(bh, s, d) * scale
    k = jnp.transpose(K, (0, 2, 1, 3)).reshape(bh, s, d)
    v = jnp.transpose(V, (0, 2, 1, 3)).reshape(bh, s, d)
    kv = jnp.concatenate([k, v], axis=-1)
    qp = jnp.concatenate([q, jnp.zeros_like(q)], -1)

    def body(q_ref, kv_ref, out_ref,
             kvc, acc, m_ref, l_ref,
             send, recv):
        my_pos = lax.axis_index("i")
        left = lax.rem(my_pos + N_DEV - 1, N_DEV)
        right = lax.rem(my_pos + 1, N_DEV)

        barrier = pltpu.get_barrier_semaphore()
        for nbr in (left, right):
            pl.semaphore_signal(
                barrier, inc=1,
                device_id=(nbr,), device_id_type=pl.DeviceIdType.MESH,
            )
        pl.semaphore_wait(barrier, 2)

        m_ref[...] = jnp.full((bh, s), -1e30, dtype=jnp.float32)
        l_ref[...] = jnp.zeros((bh, s), dtype=jnp.float32)
        acc[...] = jnp.zeros((bh, s, 2 * d), dtype=jnp.float32)

        kvc[0] = kv_ref[...]

        def flash_chunk(slot):
            def bh_body(i, _):
                qi = q_ref[i]
                kvi = kvc[slot, i]
                sij = lax.dot_general(
                    qi, kvi, (((1,), (1,)), ((), ())),
                    preferred_element_type=jnp.float32,
                )
                m_old = m_ref[i]
                m_new = jnp.maximum(m_old, jnp.max(sij, axis=1))
                p = jnp.exp(sij - m_new[:, None])
                alpha = jnp.exp(m_old - m_new)
                l_ref[i] = l_ref[i] * alpha + jnp.sum(p, axis=1)
                pkv = lax.dot_general(
                    p, kvi, (((1,), (0,)), ((), ())),
                    preferred_element_type=jnp.float32,
                )
                acc[i] = acc[i] * alpha[:, None] + pkv
                m_ref[i] = m_new
                return 0
            lax.fori_loop(0, bh, bh_body, 0)

        for hop in range(N_DEV - 1):
            cur = hop % 2
            nxt = (hop + 1) % 2
            rdma = pltpu.make_async_remote_copy(
                src_ref=kvc.at[cur], dst_ref=kvc.at[nxt],
                send_sem=send.at[cur], recv_sem=recv.at[nxt],
                device_id=(right,), device_id_type=pl.DeviceIdType.MESH,
            )
            rdma.start()
            flash_chunk(cur)
            rdma.wait()
        flash_chunk((N_DEV - 1) % 2)

        def norm_body(i, _):
            out_ref[i] = acc[i, :, d:] / l_ref[i][:, None]
            return 0
        lax.fori_loop(0, bh, norm_body, 0)

    out = pl.pallas_call(
        body,
        out_shape=jax.ShapeDtypeStruct((bh, s, d), jnp.float32),
        in_specs=[pl.BlockSpec(memory_space=pltpu.VMEM)] * 2,
        out_specs=pl.BlockSpec(memory_space=pltpu.VMEM),
        scratch_shapes=[
            pltpu.VMEM((2, bh, s, 2 * d), jnp.float32),
            pltpu.VMEM((bh, s, 2 * d), jnp.float32),
            pltpu.VMEM((bh, s), jnp.float32),
            pltpu.VMEM((bh, s), jnp.float32),
            pltpu.SemaphoreType.DMA((2,)),
            pltpu.SemaphoreType.DMA((2,)),
        ],
        compiler_params=pltpu.CompilerParams(
            collective_id=0,
            vmem_limit_bytes=100 * 1024 * 1024,
        ),
    )(qp, kv)

    return jnp.transpose(out.reshape(b, h, s, d), (0, 2, 1, 3))
